# speedup vs baseline: 1.2325x; 1.2325x over previous
import jax
import jax.numpy as jnp
from jax.experimental import pallas as pl
from jax.experimental.pallas import tpu as pltpu

DIM = 1024
N_EXPERTS = 8
MOE_INTER = 512
N_GROUPS = N_EXPERTS + 2
T = 2048
BT = 1024


def _moe_body(x_ref, gw_ref, w1_ref, w3_ref, w2_ref, out_ref, acc_ref):
    e = pl.program_id(1)
    x = x_ref[...]

    logits = jnp.dot(x, gw_ref[...], preferred_element_type=jnp.float32)
    scores = jax.nn.softmax(logits, axis=-1)
    lane = jax.lax.broadcasted_iota(jnp.int32, scores.shape, 1)
    m1 = jnp.argmax(scores, axis=-1)
    masked = jnp.where(lane == m1[:, None], -jnp.inf, scores)
    m2 = jnp.argmax(masked, axis=-1)

    sel = (lane == m1[:, None]) | (lane == m2[:, None])
    here = lane == e
    score_e = jnp.sum(jnp.where(here & sel, scores, 0.0), axis=-1)
    cw = jnp.where(e < N_EXPERTS, score_e, 1.0)

    w1 = w1_ref[0]
    w3 = w3_ref[0]
    w2 = w2_ref[0]
    a = jnp.dot(x, w1, preferred_element_type=jnp.float32)
    b = jnp.dot(x, w3, preferred_element_type=jnp.float32)
    h = jax.nn.silu(a) * b
    contrib = jnp.dot(h, w2, preferred_element_type=jnp.float32) * cw[:, None]

    @pl.when(e == 0)
    def _init():
        acc_ref[...] = contrib

    @pl.when(e > 0)
    def _acc():
        acc_ref[...] += contrib

    @pl.when(e == N_GROUPS - 1)
    def _out():
        out_ref[...] = acc_ref[...]


@jax.jit
def _moe(xf, gate_w, w1s, w3s, w2s):
    grid = (T // BT, N_GROUPS)
    return pl.pallas_call(
        _moe_body,
        grid=grid,
        in_specs=[
            pl.BlockSpec((BT, DIM), lambda t, e: (t, 0)),
            pl.BlockSpec((DIM, N_EXPERTS), lambda t, e: (0, 0)),
            pl.BlockSpec((1, DIM, MOE_INTER), lambda t, e: (e, 0, 0)),
            pl.BlockSpec((1, DIM, MOE_INTER), lambda t, e: (e, 0, 0)),
            pl.BlockSpec((1, MOE_INTER, DIM), lambda t, e: (e, 0, 0)),
        ],
        out_specs=pl.BlockSpec((BT, DIM), lambda t, e: (t, 0)),
        out_shape=jax.ShapeDtypeStruct((T, DIM), jnp.float32),
        scratch_shapes=[pltpu.VMEM((BT, DIM), jnp.float32)],
    )(xf, gate_w, w1s, w3s, w2s)


def kernel(x, gate_w, w1, w2, w3, sw1, sw2, sw3):
    shape = x.shape
    xf = x.reshape(-1, DIM)
    w1s = jnp.concatenate([w1, sw1.reshape(DIM, 2, MOE_INTER).transpose(1, 0, 2)], 0)
    w3s = jnp.concatenate([w3, sw3.reshape(DIM, 2, MOE_INTER).transpose(1, 0, 2)], 0)
    w2s = jnp.concatenate([w2, sw2.reshape(2, MOE_INTER, DIM)], 0)
    y = _moe(xf, gate_w, w1s, w3s, w2s)
    return y.reshape(shape)


# speedup vs baseline: 1.2992x; 1.0542x over previous
import jax
import jax.numpy as jnp
from jax.experimental import pallas as pl
from jax.experimental.pallas import tpu as pltpu

DIM = 1024
N_EXPERTS = 8
MOE_INTER = 512
N_GROUPS = N_EXPERTS + 2
T = 2048
BT = 1024


def _moe_body(x_ref, gw_ref, w1_ref, w3_ref, w2_ref, out_ref, acc_ref):
    e = pl.program_id(1)
    x = x_ref[...]

    logits = jnp.dot(x, gw_ref[...], preferred_element_type=jnp.float32)
    scores = jax.nn.softmax(logits, axis=-1)
    lane = jax.lax.broadcasted_iota(jnp.int32, scores.shape, 1)
    m1 = jnp.argmax(scores, axis=-1)
    masked = jnp.where(lane == m1[:, None], -jnp.inf, scores)
    m2 = jnp.argmax(masked, axis=-1)

    sel = (lane == m1[:, None]) | (lane == m2[:, None])
    here = lane == e
    score_e = jnp.sum(jnp.where(here & sel, scores, 0.0), axis=-1)
    cw = jnp.where(e < N_EXPERTS, score_e, 1.0)

    w1 = w1_ref[0]
    w3 = w3_ref[0]
    w2 = w2_ref[0]
    xb = x.astype(jnp.bfloat16)
    a = jnp.dot(xb, w1, preferred_element_type=jnp.float32)
    b = jnp.dot(xb, w3, preferred_element_type=jnp.float32)
    h = (jax.nn.silu(a) * b).astype(jnp.bfloat16)
    contrib = jnp.dot(h, w2, preferred_element_type=jnp.float32) * cw[:, None]

    @pl.when(e == 0)
    def _init():
        acc_ref[...] = contrib

    @pl.when(e > 0)
    def _acc():
        acc_ref[...] += contrib

    @pl.when(e == N_GROUPS - 1)
    def _out():
        out_ref[...] = acc_ref[...]


@jax.jit
def _moe(xf, gate_w, w1s, w3s, w2s):
    grid = (T // BT, N_GROUPS)
    return pl.pallas_call(
        _moe_body,
        grid=grid,
        in_specs=[
            pl.BlockSpec((BT, DIM), lambda t, e: (t, 0)),
            pl.BlockSpec((DIM, N_EXPERTS), lambda t, e: (0, 0)),
            pl.BlockSpec((1, DIM, MOE_INTER), lambda t, e: (e, 0, 0)),
            pl.BlockSpec((1, DIM, MOE_INTER), lambda t, e: (e, 0, 0)),
            pl.BlockSpec((1, MOE_INTER, DIM), lambda t, e: (e, 0, 0)),
        ],
        out_specs=pl.BlockSpec((BT, DIM), lambda t, e: (t, 0)),
        out_shape=jax.ShapeDtypeStruct((T, DIM), jnp.float32),
        scratch_shapes=[pltpu.VMEM((BT, DIM), jnp.float32)],
    )(xf, gate_w, w1s, w3s, w2s)


def kernel(x, gate_w, w1, w2, w3, sw1, sw2, sw3):
    shape = x.shape
    xf = x.reshape(-1, DIM)
    w1s = jnp.concatenate(
        [w1, sw1.reshape(DIM, 2, MOE_INTER).transpose(1, 0, 2)], 0
    ).astype(jnp.bfloat16)
    w3s = jnp.concatenate(
        [w3, sw3.reshape(DIM, 2, MOE_INTER).transpose(1, 0, 2)], 0
    ).astype(jnp.bfloat16)
    w2s = jnp.concatenate([w2, sw2.reshape(2, MOE_INTER, DIM)], 0).astype(jnp.bfloat16)
    y = _moe(xf, gate_w, w1s, w3s, w2s)
    return y.reshape(shape)
